# baseline (speedup 1.0000x reference)
import jax
import jax.numpy as jnp
from jax import lax
from jax.experimental import pallas as pl
from jax.experimental.pallas import tpu as pltpu
from jax.experimental.pallas import tpu_sc as plsc

D = 64
S = 200
NW = 32
BBLK = 128
LANES = 16
SPU = 2
NU = S // SPU
NGB = 3
NTB = 2


def _sc_body(ids_hbm, tok_hbm, pos_hbm, out_hbm, idx_v, pos_v, *bufs_sems):
    gbufs = bufs_sems[0:NGB]
    tbufs = bufs_sems[NGB:NGB + NTB]
    gsems = bufs_sems[NGB + NTB:2 * NGB + NTB]
    ssems = bufs_sems[2 * NGB + NTB:2 * NGB + 2 * NTB]

    wid = lax.axis_index("s") * 2 + lax.axis_index("c")

    pltpu.sync_copy(ids_hbm.at[pl.ds(0, S // 8), wid], idx_v)
    pltpu.sync_copy(pos_hbm.at[pl.ds(0, S)], pos_v)

    iota = lax.iota(jnp.int32, LANES)
    rows = [jnp.full((LANES,), j * LANES, jnp.int32) + iota for j in range(8)]

    def start_gather(u, b):
        st = u // (8 // SPU)
        slq = lax.rem(u, 8 // SPU)
        for h in range(SPU * BBLK // LANES):
            iv = idx_v[st, slq, pl.ds(h * LANES, LANES)]
            pltpu.async_copy(
                tok_hbm.at[iv],
                gbufs[b].at[pl.ds(h * LANES, LANES)], gsems[b])

    def wait_gather(b):
        pltpu.make_async_copy(tok_hbm.at[pl.ds(0, SPU * BBLK)],
                              gbufs[b], gsems[b]).wait()

    def start_scatter(u, b):
        pltpu.async_copy(tbufs[b],
                         out_hbm.at[pl.ds(u * SPU, SPU), pl.ds(0, D // 8), wid],
                         ssems[b])

    def wait_scatter(b):
        pltpu.make_async_copy(tbufs[b],
                              out_hbm.at[pl.ds(0, SPU), pl.ds(0, D // 8), 0],
                              ssems[b]).wait()

    def build(u, gb, tb):
        for q in range(SPU):
            s16 = jnp.full((LANES,), u * SPU + q, jnp.int32)
            gref = gbufs[gb].at[pl.ds(q * BBLK, BBLK)]
            tref = tbufs[tb].at[q]

            @plsc.parallel_loop(0, D, 1, unroll=8)
            def d_body(dbase):
                dcol = (iota + dbase) & (D - 1)
                p = plsc.load_gather(pos_v, [s16, dcol])
                dh = lax.shift_right_logical(dcol, 3)
                dl = dcol & 7
                for j in range(8):
                    x = plsc.load_gather(gref, [rows[j], dcol])
                    plsc.store_scatter(tref, [dh, dl, rows[j]], x + p)

    for u0 in range(NGB - 1):
        start_gather(u0, u0)

    nstep = NGB * NTB

    def outer(i, carry):
        for k in range(nstep):
            u = i * nstep + k
            gb = k % NGB
            tb = k % NTB

            @pl.when(u + NGB - 1 < NU)
            def _():
                start_gather(u + NGB - 1, (k + NGB - 1) % NGB)

            wait_gather(gb)

            @pl.when(u >= NTB)
            def _():
                wait_scatter(tb)

            build(u, gb, tb)
            start_scatter(u, tb)
        return carry

    lax.fori_loop(0, NU // nstep, outer, 0)

    for k in range(NU % nstep):
        u = (NU // nstep) * nstep + k
        gb = u % NGB
        tb = u % NTB

        @pl.when(u + NGB - 1 < NU)
        def _():
            start_gather(u + NGB - 1, (u + NGB - 1) % NGB)

        wait_gather(gb)
        wait_scatter(tb)
        build(u, gb, tb)
        start_scatter(u, tb)

    for b in range(NTB):
        wait_scatter(b)


def kernel(input_ids, tok_table, pos_table):
    bsz, seq_len = input_ids.shape
    ids4 = input_ids.reshape(bsz // BBLK, BBLK, seq_len // 8, 8)
    ids4 = ids4.transpose(2, 0, 3, 1).reshape(seq_len // 8, bsz // BBLK,
                                              8 // SPU, SPU * BBLK)

    mesh = plsc.VectorSubcoreMesh(core_axis_name="c", subcore_axis_name="s")
    run = pl.kernel(
        _sc_body, mesh=mesh,
        out_type=jax.ShapeDtypeStruct((S, D // 8, NW, 8, BBLK), jnp.float32),
        scratch_types=(
            [pltpu.VMEM((S // 8, 8 // SPU, SPU * BBLK), jnp.int32)]
            + [pltpu.VMEM((S, D), jnp.float32)]
            + [pltpu.VMEM((SPU * BBLK, D), jnp.float32) for _ in range(NGB)]
            + [pltpu.VMEM((SPU, D // 8, 8, BBLK), jnp.float32)
               for _ in range(NTB)]
            + [pltpu.SemaphoreType.DMA for _ in range(NGB + NTB)]
        ),
        compiler_params=pltpu.CompilerParams(use_tc_tiling_on_sc=False,
                                             needs_layout_passes=False),
    )
    out5 = run(ids4, tok_table, pos_table)
    out = out5.transpose(2, 4, 0, 1, 3).reshape(bsz, seq_len, D)
    return out


# speedup vs baseline: 1.0363x; 1.0363x over previous
import jax
import jax.numpy as jnp
from jax import lax
from jax.experimental import pallas as pl
from jax.experimental.pallas import tpu as pltpu
from jax.experimental.pallas import tpu_sc as plsc

D = 64
S = 200
NW = 32
BBLK = 128
LANES = 16
NBUF = 5


def _sc_body(ids_hbm, tok_hbm, pos_hbm, out_hbm, idx_v, pos_v, *bufs_sems):
    gbufs = bufs_sems[0:NBUF]
    tbufs = bufs_sems[NBUF:2 * NBUF]
    gsems = bufs_sems[2 * NBUF:3 * NBUF]
    ssems = bufs_sems[3 * NBUF:4 * NBUF]

    wid = lax.axis_index("s") * 2 + lax.axis_index("c")

    pltpu.sync_copy(ids_hbm.at[pl.ds(0, S // 8), wid], idx_v)
    pltpu.sync_copy(pos_hbm.at[pl.ds(0, S)], pos_v)

    iota = lax.iota(jnp.int32, LANES)
    rows = [jnp.full((LANES,), j * LANES, jnp.int32) + iota for j in range(8)]

    def start_gather(s, b):
        pltpu.async_copy(tok_hbm.at[idx_v.at[s // 8, s % 8]], gbufs[b],
                         gsems[b])

    def wait_gather(b):
        pltpu.make_async_copy(tok_hbm.at[pl.ds(0, BBLK)], gbufs[b],
                              gsems[b]).wait()

    def start_scatter(s, b):
        pltpu.async_copy(tbufs[b], out_hbm.at[s, pl.ds(0, D // 8), wid],
                         ssems[b])

    def wait_scatter(b):
        pltpu.make_async_copy(tbufs[b], out_hbm.at[0, pl.ds(0, D // 8), 0],
                              ssems[b]).wait()

    def build(s, b):
        s16 = jnp.full((LANES,), s, jnp.int32)

        @plsc.parallel_loop(0, D, 1, unroll=8)
        def d_body(dbase):
            dcol = (iota + dbase) & (D - 1)
            p = plsc.load_gather(pos_v, [s16, dcol])
            dh = lax.shift_right_logical(dcol, 3)
            dl = dcol & 7
            for j in range(8):
                x = plsc.load_gather(gbufs[b], [rows[j], dcol])
                plsc.store_scatter(tbufs[b], [dh, dl, rows[j]], x + p)

    for s0 in range(NBUF - 1):
        start_gather(s0, s0)

    def outer(i, carry):
        for b in range(NBUF):
            s = i * NBUF + b

            @pl.when(s + NBUF - 1 < S)
            def _():
                start_gather(s + NBUF - 1, (b + NBUF - 1) % NBUF)

            wait_gather(b)

            @pl.when(s >= NBUF)
            def _():
                wait_scatter(b)

            build(s, b)
            start_scatter(s, b)
        return carry

    lax.fori_loop(0, S // NBUF, outer, 0)
    for b in range(NBUF):
        wait_scatter(b)


def kernel(input_ids, tok_table, pos_table):
    bsz, seq_len = input_ids.shape
    ids4 = input_ids.reshape(bsz // BBLK, BBLK, seq_len // 8, 8)
    ids4 = ids4.transpose(2, 0, 3, 1)

    mesh = plsc.VectorSubcoreMesh(core_axis_name="c", subcore_axis_name="s")
    run = pl.kernel(
        _sc_body, mesh=mesh,
        out_type=jax.ShapeDtypeStruct((S, D // 8, NW, 8, BBLK), jnp.float32),
        scratch_types=(
            [pltpu.VMEM((S // 8, 8, BBLK), jnp.int32)]
            + [pltpu.VMEM((S, D), jnp.float32)]
            + [pltpu.VMEM((BBLK, D), jnp.float32) for _ in range(NBUF)]
            + [pltpu.VMEM((D // 8, 8, BBLK), jnp.float32) for _ in range(NBUF)]
            + [pltpu.SemaphoreType.DMA for _ in range(2 * NBUF)]
        ),
        compiler_params=pltpu.CompilerParams(use_tc_tiling_on_sc=False,
                                             needs_layout_passes=False),
    )
    out5 = run(ids4, tok_table, pos_table)
    out = out5.transpose(2, 4, 0, 1, 3).reshape(bsz, seq_len, D)
    return out
